# Initial kernel scaffold; baseline (speedup 1.0000x reference)
import functools

import jax
import jax.numpy as jnp
from jax import lax
from jax.experimental import pallas as pl
from jax.experimental.pallas import tpu as pltpu
from jax.experimental.pallas import tpu_sc as plsc

N = 10000
D = 128
E = 320000
NS = 16
CH = 128
ET = E // NS
NCH = ET // CH
TAIL = ET - NCH * CH
RPT = N // NS
BLK = 1000

_mesh = plsc.VectorSubcoreMesh(core_axis_name="c", subcore_axis_name="s")


@functools.partial(
    pl.kernel,
    mesh=_mesh,
    out_type=jax.ShapeDtypeStruct((2, N, 16), jnp.float32),
    scratch_types=[
        pltpu.VMEM((CH,), jnp.int32),
        pltpu.VMEM((TAIL,), jnp.int32),
        pltpu.VMEM((CH, 16), jnp.float32),
        pltpu.VMEM_SHARED((N, 16), jnp.float32),
        pltpu.SemaphoreType.DMA,
    ],
)
def _deg_kernel(dst_ref, zeros_ref, ones_ref, out_ref,
                idx_v, idx_t, ones_v, acc_s, sem):
    c = lax.axis_index("c")
    s = lax.axis_index("s")
    pltpu.sync_copy(ones_ref, ones_v)
    pltpu.sync_copy(zeros_ref.at[pl.ds(s * RPT, RPT)],
                    acc_s.at[pl.ds(s * RPT, RPT)])
    plsc.subcore_barrier()
    base = s * ET

    @pl.loop(0, NCH)
    def _(k):
        pltpu.sync_copy(dst_ref.at[c, pl.ds(base + k * CH, CH)], idx_v)
        pltpu.sync_copy(ones_v, acc_s.at[idx_v], add=True)

    pltpu.sync_copy(dst_ref.at[c, pl.ds(base + NCH * CH, TAIL)], idx_t)
    pltpu.sync_copy(ones_v.at[pl.ds(0, TAIL)], acc_s.at[idx_t], add=True)
    plsc.subcore_barrier()
    pltpu.sync_copy(acc_s.at[pl.ds(s * RPT, RPT)],
                    out_ref.at[c, pl.ds(s * RPT, RPT)])


@functools.partial(
    pl.kernel,
    mesh=_mesh,
    out_type=jax.ShapeDtypeStruct((2, N, D), jnp.float32),
    scratch_types=[
        pltpu.VMEM((CH,), jnp.int32),
        pltpu.VMEM((CH,), jnp.int32),
        pltpu.VMEM((CH, D), jnp.float32),
        pltpu.VMEM((TAIL,), jnp.int32),
        pltpu.VMEM((TAIL,), jnp.int32),
        pltpu.VMEM((TAIL, D), jnp.float32),
        pltpu.VMEM_SHARED((N, D), jnp.float32),
        pltpu.SemaphoreType.DMA,
    ],
)
def _agg_kernel(y_ref, srcoff_ref, dst_ref, out_ref,
                sidx_v, didx_v, rows_v, sidx_t, didx_t, rows_t, acc_s, sem):
    c = lax.axis_index("c")
    s = lax.axis_index("s")
    pltpu.sync_copy(y_ref.at[pl.ds(c * N + s * RPT, RPT)],
                    acc_s.at[pl.ds(s * RPT, RPT)])
    plsc.subcore_barrier()
    base = s * ET

    @pl.loop(0, NCH)
    def _(k):
        pltpu.sync_copy(srcoff_ref.at[c, pl.ds(base + k * CH, CH)], sidx_v)
        pltpu.sync_copy(dst_ref.at[c, pl.ds(base + k * CH, CH)], didx_v)
        pltpu.async_copy(y_ref.at[sidx_v], rows_v, sem).wait()
        pltpu.sync_copy(rows_v, acc_s.at[didx_v], add=True)

    pltpu.sync_copy(srcoff_ref.at[c, pl.ds(base + NCH * CH, TAIL)], sidx_t)
    pltpu.sync_copy(dst_ref.at[c, pl.ds(base + NCH * CH, TAIL)], didx_t)
    pltpu.async_copy(y_ref.at[sidx_t], rows_t, sem).wait()
    pltpu.sync_copy(rows_t, acc_s.at[didx_t], add=True)
    plsc.subcore_barrier()
    pltpu.sync_copy(acc_s.at[pl.ds(s * RPT, RPT)],
                    out_ref.at[c, pl.ds(s * RPT, RPT)])


def _scale_body(x_ref, w_ref, deg_ref, y_ref):
    xw = jnp.dot(x_ref[...], w_ref[...], preferred_element_type=jnp.float32)
    d0 = lax.rsqrt(deg_ref[0, :, 0:1] + 1.0)
    d1 = lax.rsqrt(deg_ref[1, :, 0:1] + 1.0)
    y_ref[0] = d0 * xw
    y_ref[1] = d1 * xw


_scale_call = pl.pallas_call(
    _scale_body,
    grid=(N // BLK,),
    in_specs=[
        pl.BlockSpec((BLK, D), lambda i: (i, 0)),
        pl.BlockSpec((D, D), lambda i: (0, 0)),
        pl.BlockSpec((2, BLK, 16), lambda i: (0, i, 0)),
    ],
    out_specs=pl.BlockSpec((2, BLK, D), lambda i: (0, i, 0)),
    out_shape=jax.ShapeDtypeStruct((2, N, D), jnp.float32),
)


def _final_body(agg_ref, deg_ref, b_ref, out_ref):
    bv = b_ref[0]
    d0 = lax.rsqrt(deg_ref[0, :, 0:1] + 1.0)
    d1 = lax.rsqrt(deg_ref[1, :, 0:1] + 1.0)
    out_ref[:, :D] = d0 * agg_ref[0] + bv
    out_ref[:, D:] = d1 * agg_ref[1] + bv


_final_call = pl.pallas_call(
    _final_body,
    grid=(N // BLK,),
    in_specs=[
        pl.BlockSpec((2, BLK, D), lambda i: (0, i, 0)),
        pl.BlockSpec((2, BLK, 16), lambda i: (0, i, 0)),
        pl.BlockSpec((1, D), lambda i: (0, 0)),
    ],
    out_specs=pl.BlockSpec((BLK, 2 * D), lambda i: (i, 0)),
    out_shape=jax.ShapeDtypeStruct((N, 2 * D), jnp.float32),
)


def kernel(x, edge_index_list, W, b):
    ei = edge_index_list.astype(jnp.int32)
    src = ei[:, 0, :]
    dst = ei[:, 1, :]
    srcoff = src + jnp.arange(2, dtype=jnp.int32)[:, None] * N
    zeros16 = jnp.zeros((N, 16), jnp.float32)
    ones16 = jnp.ones((CH, 16), jnp.float32)

    degp = _deg_kernel(dst, zeros16, ones16)
    y = _scale_call(x, W, degp)
    agg = _agg_kernel(y.reshape(2 * N, D), srcoff, dst)
    return _final_call(agg, degp, b.reshape(1, D))


# speedup vs baseline: 17.8838x; 17.8838x over previous
import functools

import jax
import jax.numpy as jnp
from jax import lax
from jax.experimental import pallas as pl
from jax.experimental.pallas import tpu as pltpu
from jax.experimental.pallas import tpu_sc as plsc

N = 10000
D = 128
E = 320000
NS = 16
CH = 128
ET = E // NS
NCH = ET // CH
TAIL = ET - NCH * CH
RPT = (N // NS) // 8 * 8
RTL = N - NS * RPT
BLK = 1000


def _deg_body(dst_ref, zeros_ref, ones_ref, out_ref,
              idx_v, idx_t, ones_v, acc_s, sem):
    c = lax.axis_index("c")
    s = lax.axis_index("s")
    pltpu.sync_copy(ones_ref, ones_v)
    pltpu.sync_copy(zeros_ref.at[pl.ds(s * RPT, RPT)],
                    acc_s.at[pl.ds(s * RPT, RPT)])

    @pl.when(s == NS - 1)
    def _():
        pltpu.sync_copy(zeros_ref.at[pl.ds(NS * RPT, RTL)],
                        acc_s.at[pl.ds(NS * RPT, RTL)])

    plsc.subcore_barrier()
    base = c * E + s * ET

    @pl.loop(0, NCH)
    def _(k):
        pltpu.sync_copy(dst_ref.at[pl.ds(base + k * CH, CH)], idx_v)
        pltpu.sync_copy(ones_v, acc_s.at[idx_v], add=True)

    pltpu.sync_copy(dst_ref.at[pl.ds(base + NCH * CH, TAIL)], idx_t)
    pltpu.sync_copy(ones_v.at[pl.ds(0, TAIL)], acc_s.at[idx_t], add=True)
    plsc.subcore_barrier()
    pltpu.sync_copy(acc_s.at[pl.ds(s * RPT, RPT)],
                    out_ref.at[c, pl.ds(s * RPT, RPT)])

    @pl.when(s == NS - 1)
    def _():
        pltpu.sync_copy(acc_s.at[pl.ds(NS * RPT, RTL)],
                        out_ref.at[c, pl.ds(NS * RPT, RTL)])


def _agg_body(y_ref, srcoff_ref, dst_ref, out_ref,
              sidx_v, didx_v, rows_v, sidx_t, didx_t, rows_t, acc_s, sem):
    c = lax.axis_index("c")
    s = lax.axis_index("s")
    pltpu.sync_copy(y_ref.at[pl.ds(c * N + s * RPT, RPT)],
                    acc_s.at[pl.ds(s * RPT, RPT)])

    @pl.when(s == NS - 1)
    def _():
        pltpu.sync_copy(y_ref.at[pl.ds(c * N + NS * RPT, RTL)],
                        acc_s.at[pl.ds(NS * RPT, RTL)])

    plsc.subcore_barrier()
    base = c * E + s * ET

    @pl.loop(0, NCH)
    def _(k):
        pltpu.sync_copy(srcoff_ref.at[pl.ds(base + k * CH, CH)], sidx_v)
        pltpu.sync_copy(dst_ref.at[pl.ds(base + k * CH, CH)], didx_v)
        pltpu.async_copy(y_ref.at[sidx_v], rows_v, sem).wait()
        pltpu.sync_copy(rows_v, acc_s.at[didx_v], add=True)

    pltpu.sync_copy(srcoff_ref.at[pl.ds(base + NCH * CH, TAIL)], sidx_t)
    pltpu.sync_copy(dst_ref.at[pl.ds(base + NCH * CH, TAIL)], didx_t)
    pltpu.async_copy(y_ref.at[sidx_t], rows_t, sem).wait()
    pltpu.sync_copy(rows_t, acc_s.at[didx_t], add=True)
    plsc.subcore_barrier()
    pltpu.sync_copy(acc_s.at[pl.ds(s * RPT, RPT)],
                    out_ref.at[c, pl.ds(s * RPT, RPT)])

    @pl.when(s == NS - 1)
    def _():
        pltpu.sync_copy(acc_s.at[pl.ds(NS * RPT, RTL)],
                        out_ref.at[c, pl.ds(NS * RPT, RTL)])


def _scale_body(x_ref, w_ref, deg_ref, y_ref):
    xw = jnp.dot(x_ref[...], w_ref[...], preferred_element_type=jnp.float32)
    d0 = lax.rsqrt(deg_ref[0, :, 0:1] + 1.0)
    d1 = lax.rsqrt(deg_ref[1, :, 0:1] + 1.0)
    y_ref[0] = d0 * xw
    y_ref[1] = d1 * xw


_scale_call = pl.pallas_call(
    _scale_body,
    grid=(N // BLK,),
    in_specs=[
        pl.BlockSpec((BLK, D), lambda i: (i, 0)),
        pl.BlockSpec((D, D), lambda i: (0, 0)),
        pl.BlockSpec((2, BLK, D), lambda i: (0, i, 0)),
    ],
    out_specs=pl.BlockSpec((2, BLK, D), lambda i: (0, i, 0)),
    out_shape=jax.ShapeDtypeStruct((2, N, D), jnp.float32),
)


def _final_body(agg_ref, deg_ref, b_ref, out_ref):
    bv = b_ref[0]
    d0 = lax.rsqrt(deg_ref[0, :, 0:1] + 1.0)
    d1 = lax.rsqrt(deg_ref[1, :, 0:1] + 1.0)
    out_ref[:, :D] = d0 * agg_ref[0] + bv
    out_ref[:, D:] = d1 * agg_ref[1] + bv


_final_call = pl.pallas_call(
    _final_body,
    grid=(N // BLK,),
    in_specs=[
        pl.BlockSpec((2, BLK, D), lambda i: (0, i, 0)),
        pl.BlockSpec((2, BLK, D), lambda i: (0, i, 0)),
        pl.BlockSpec((1, D), lambda i: (0, 0)),
    ],
    out_specs=pl.BlockSpec((BLK, 2 * D), lambda i: (i, 0)),
    out_shape=jax.ShapeDtypeStruct((N, 2 * D), jnp.float32),
)


@functools.cache
def _sc_kernels():
    mesh = plsc.VectorSubcoreMesh(core_axis_name="c", subcore_axis_name="s")
    deg_kernel = pl.kernel(
        _deg_body,
        mesh=mesh,
        out_type=jax.ShapeDtypeStruct((2, N, D), jnp.float32),
        scratch_types=[
            pltpu.VMEM((CH,), jnp.int32),
            pltpu.VMEM((TAIL,), jnp.int32),
            pltpu.VMEM((CH, D), jnp.float32),
            pltpu.VMEM_SHARED((N, D), jnp.float32),
            pltpu.SemaphoreType.DMA,
        ],
    )
    agg_kernel = pl.kernel(
        _agg_body,
        mesh=mesh,
        out_type=jax.ShapeDtypeStruct((2, N, D), jnp.float32),
        scratch_types=[
            pltpu.VMEM((CH,), jnp.int32),
            pltpu.VMEM((CH,), jnp.int32),
            pltpu.VMEM((CH, D), jnp.float32),
            pltpu.VMEM((TAIL,), jnp.int32),
            pltpu.VMEM((TAIL,), jnp.int32),
            pltpu.VMEM((TAIL, D), jnp.float32),
            pltpu.VMEM_SHARED((N, D), jnp.float32),
            pltpu.SemaphoreType.DMA,
        ],
    )
    return deg_kernel, agg_kernel


def kernel(x, edge_index_list, W, b):
    deg_kernel, agg_kernel = _sc_kernels()
    ei = edge_index_list.astype(jnp.int32)
    src = ei[:, 0, :]
    dst = ei[:, 1, :].reshape(2 * E)
    srcoff = (src + jnp.arange(2, dtype=jnp.int32)[:, None] * N).reshape(2 * E)
    zerosd = jnp.zeros((N, D), jnp.float32)
    onesd = jnp.ones((CH, D), jnp.float32)

    degp = deg_kernel(dst, zerosd, onesd)
    y = _scale_call(x, W, degp)
    agg = agg_kernel(y.reshape(2 * N, D), srcoff, dst)
    return _final_call(agg, degp, b.reshape(1, D))
